# baseline (speedup 1.0000x reference)
import functools

import jax
import jax.numpy as jnp
from jax import lax
from jax.experimental import pallas as pl
from jax.experimental.pallas import tpu as pltpu
from jax.experimental.pallas import tpu_sc as plsc

NC = 2
NS = 16
L = 16
B = 16384
K = 16
BPW = B // (NC * NS)
NBLK = BPW // L
SCALE = 0.5 / B


K2 = K // 2


def _splfm_body(*refs):
  (sampleU, sampleI, sampleF, sampleR, alpha_arr,
   betaU, betaI, thetaU) = refs[:8]
  gU = refs[8:8 + K2]
  gI = refs[8 + K2:8 + 2 * K2]
  (out_hbm, idxU_v, idxI_v, f_v, r_v, alpha_v,
   bu_v, bi_v, tu_v, gup_v, gip_v,
   lacc_v, out_v, allp_v, shared, sem) = refs[8 + 2 * K2:]

  cid = lax.axis_index("c")
  sid = lax.axis_index("s")
  wid = sid * NC + cid
  base = wid * BPW

  pltpu.sync_copy(sampleU.at[pl.ds(base, BPW)], idxU_v)
  pltpu.sync_copy(sampleI.at[pl.ds(base, BPW)], idxI_v)
  pltpu.sync_copy(sampleF.at[pl.ds(base, BPW)], f_v)
  pltpu.sync_copy(sampleR.at[pl.ds(base, BPW)], r_v)
  pltpu.sync_copy(alpha_arr, alpha_v)

  cps = [
      pltpu.async_copy(betaU.at[idxU_v], bu_v, sem),
      pltpu.async_copy(betaI.at[idxI_v], bi_v, sem),
      pltpu.async_copy(thetaU.at[idxU_v], tu_v, sem),
  ]
  for k in range(K2):
    dst = pl.ds(k * BPW, BPW)
    cps.append(pltpu.async_copy(gU[k].at[idxU_v], gup_v.at[dst], sem))
    cps.append(pltpu.async_copy(gI[k].at[idxI_v], gip_v.at[dst], sem))
  for cp in cps:
    cp.wait()

  av = alpha_v[...]
  himask = jnp.full((L,), 0xFFFF0000, jnp.uint32)
  sh16 = jnp.full((L,), 16, jnp.uint32)

  def block_body(blk, acc):
    r0 = blk * L
    bu = bu_v[pl.ds(r0, L)]
    bi = bi_v[pl.ds(r0, L)]
    tu = tu_v[pl.ds(r0, L)]
    f = f_v[pl.ds(r0, L)]
    r = r_v[pl.ds(r0, L)]
    dot = jnp.zeros((L,), jnp.float32)
    for k in range(K2):
      wu = plsc.bitcast(gup_v[pl.ds(k * BPW + r0, L)], jnp.uint32)
      wi = plsc.bitcast(gip_v[pl.ds(k * BPW + r0, L)], jnp.uint32)
      u_hi = plsc.bitcast(wu & himask, jnp.float32)
      i_hi = plsc.bitcast(wi & himask, jnp.float32)
      u_lo = plsc.bitcast(wu << sh16, jnp.float32)
      i_lo = plsc.bitcast(wi << sh16, jnp.float32)
      dot = dot + u_hi * i_hi + u_lo * i_lo
    e = av + bu + bi + tu * f + dot - r
    return acc + e * e

  loss_acc = lax.fori_loop(0, NBLK, block_body, jnp.zeros((L,), jnp.float32))

  lacc_v[...] = loss_acc
  pltpu.sync_copy(lacc_v, shared.at[sid])
  plsc.subcore_barrier()

  @pl.when(sid == 0)
  def _():
    pltpu.sync_copy(shared, allp_v)
    tot = jnp.zeros((L,), jnp.float32)
    for i in range(NS):
      tot = tot + allp_v[i, :]
    total = jnp.sum(tot) * SCALE
    out_v[...] = jnp.full((L,), total, jnp.float32)
    pltpu.sync_copy(out_v, out_hbm.at[pl.ds(cid * L, L)])


_splfm_call = functools.partial(
    pl.kernel,
    out_type=jax.ShapeDtypeStruct((NC * L,), jnp.float32),
    mesh=plsc.VectorSubcoreMesh(core_axis_name="c", subcore_axis_name="s"),
    compiler_params=pltpu.CompilerParams(
        needs_layout_passes=False, use_tc_tiling_on_sc=False),
    scratch_types=[
        pltpu.VMEM((BPW,), jnp.int32),
        pltpu.VMEM((BPW,), jnp.int32),
        pltpu.VMEM((BPW,), jnp.float32),
        pltpu.VMEM((BPW,), jnp.float32),
        pltpu.VMEM((L,), jnp.float32),
        pltpu.VMEM((BPW,), jnp.float32),
        pltpu.VMEM((BPW,), jnp.float32),
        pltpu.VMEM((BPW,), jnp.float32),
        pltpu.VMEM((K2 * BPW,), jnp.float32),
        pltpu.VMEM((K2 * BPW,), jnp.float32),
        pltpu.VMEM((L,), jnp.float32),
        pltpu.VMEM((L,), jnp.float32),
        pltpu.VMEM((NS, L), jnp.float32),
        pltpu.VMEM_SHARED((NS, L), jnp.float32),
        pltpu.SemaphoreType.DMA,
    ],
)(_splfm_body)


_N = 1000000
_CHUNK = 65536
_NCHUNK = -(-_N // _CHUNK)


def _detile_body(gtu_ref, gti_ref, *out_refs):
  for t, ref in enumerate((gtu_ref, gti_ref)):
    for k in range(K2):
      a = lax.bitcast_convert_type(ref[2 * k, :], jnp.uint32)
      b = lax.bitcast_convert_type(ref[2 * k + 1, :], jnp.uint32)
      packed = (a & jnp.uint32(0xFFFF0000)) | (b >> jnp.uint32(16))
      out_refs[t * K2 + k][...] = lax.bitcast_convert_type(
          packed, jnp.float32)


_detile = pl.pallas_call(
    _detile_body,
    grid=(_NCHUNK,),
    in_specs=[pl.BlockSpec((K, _CHUNK), lambda c: (0, c))] * 2,
    out_specs=[pl.BlockSpec((_CHUNK,), lambda c: (c,))] * (2 * K2),
    out_shape=[jax.ShapeDtypeStruct((_N,), jnp.float32)] * (2 * K2),
)


@jax.jit
def kernel(sampleU, sampleI, sampleF, sampleR, alpha,
           betaU, betaI, thetaU, gammaU, gammaI):
  alpha_arr = jnp.full((L,), alpha, dtype=jnp.float32)
  planes = _detile(gammaU.T, gammaI.T)
  gU = planes[:K2]
  gI = planes[K2:]
  out = _splfm_call(sampleU, sampleI, sampleF, sampleR, alpha_arr,
                    betaU, betaI, thetaU, *gU, *gI)
  return out[0] + out[L]


# speedup vs baseline: 1.2633x; 1.2633x over previous
import functools

import jax
import jax.numpy as jnp
from jax import lax
from jax.experimental import pallas as pl
from jax.experimental.pallas import tpu as pltpu
from jax.experimental.pallas import tpu_sc as plsc

NC = 2
NS = 16
L = 16
B = 16384
K = 16
BPW = B // (NC * NS)
NBLK = BPW // L
SCALE = 0.5 / B


def _splfm_body(*refs):
  (sampleU, sampleI, sampleF, sampleR, alpha_arr,
   betaU, betaI, thetaU) = refs[:8]
  gU = refs[8:8 + K]
  gI = refs[8 + K:8 + 2 * K]
  (out_hbm, idxU_v, idxI_v, f_v, r_v, alpha_v,
   bu_v, bi_v, tu_v, gup_v, gip_v,
   lacc_v, out_v, allp_v, shared, sem) = refs[8 + 2 * K:]

  cid = lax.axis_index("c")
  sid = lax.axis_index("s")
  wid = sid * NC + cid
  base = wid * BPW

  pltpu.sync_copy(sampleU.at[pl.ds(base, BPW)], idxU_v)
  pltpu.sync_copy(sampleI.at[pl.ds(base, BPW)], idxI_v)
  pltpu.sync_copy(sampleF.at[pl.ds(base, BPW)], f_v)
  pltpu.sync_copy(sampleR.at[pl.ds(base, BPW)], r_v)
  pltpu.sync_copy(alpha_arr, alpha_v)

  cps = [
      pltpu.async_copy(betaU.at[idxU_v], bu_v, sem),
      pltpu.async_copy(betaI.at[idxI_v], bi_v, sem),
      pltpu.async_copy(thetaU.at[idxU_v], tu_v, sem),
  ]
  for k in range(K):
    dst = pl.ds(k * BPW, BPW)
    cps.append(pltpu.async_copy(gU[k].at[idxU_v], gup_v.at[dst], sem))
    cps.append(pltpu.async_copy(gI[k].at[idxI_v], gip_v.at[dst], sem))
  for cp in cps:
    cp.wait()

  av = alpha_v[...]

  def block_body(blk, acc):
    r0 = blk * L
    bu = bu_v[pl.ds(r0, L)]
    bi = bi_v[pl.ds(r0, L)]
    tu = tu_v[pl.ds(r0, L)]
    f = f_v[pl.ds(r0, L)]
    r = r_v[pl.ds(r0, L)]
    dot = jnp.zeros((L,), jnp.float32)
    for k in range(K):
      dot = dot + (gup_v[pl.ds(k * BPW + r0, L)] *
                   gip_v[pl.ds(k * BPW + r0, L)])
    e = av + bu + bi + tu * f + dot - r
    return acc + e * e

  loss_acc = lax.fori_loop(0, NBLK, block_body, jnp.zeros((L,), jnp.float32))

  lacc_v[...] = loss_acc
  pltpu.sync_copy(lacc_v, shared.at[sid])
  plsc.subcore_barrier()

  @pl.when(sid == 0)
  def _():
    pltpu.sync_copy(shared, allp_v)
    tot = jnp.zeros((L,), jnp.float32)
    for i in range(NS):
      tot = tot + allp_v[i, :]
    total = jnp.sum(tot) * SCALE
    out_v[...] = jnp.full((L,), total, jnp.float32)
    pltpu.sync_copy(out_v, out_hbm.at[pl.ds(cid * L, L)])


_splfm_call = functools.partial(
    pl.kernel,
    out_type=jax.ShapeDtypeStruct((NC * L,), jnp.float32),
    mesh=plsc.VectorSubcoreMesh(core_axis_name="c", subcore_axis_name="s"),
    compiler_params=pltpu.CompilerParams(
        needs_layout_passes=False, use_tc_tiling_on_sc=False),
    scratch_types=[
        pltpu.VMEM((BPW,), jnp.int32),
        pltpu.VMEM((BPW,), jnp.int32),
        pltpu.VMEM((BPW,), jnp.float32),
        pltpu.VMEM((BPW,), jnp.float32),
        pltpu.VMEM((L,), jnp.float32),
        pltpu.VMEM((BPW,), jnp.float32),
        pltpu.VMEM((BPW,), jnp.float32),
        pltpu.VMEM((BPW,), jnp.float32),
        pltpu.VMEM((K * BPW,), jnp.float32),
        pltpu.VMEM((K * BPW,), jnp.float32),
        pltpu.VMEM((L,), jnp.float32),
        pltpu.VMEM((L,), jnp.float32),
        pltpu.VMEM((NS, L), jnp.float32),
        pltpu.VMEM_SHARED((NS, L), jnp.float32),
        pltpu.SemaphoreType.DMA,
    ],
)(_splfm_body)


_N = 1000000
_CHUNK = 65536
_NCHUNK = -(-_N // _CHUNK)


def _detile_body(gtu_ref, gti_ref, *out_refs):
  for k in range(K):
    out_refs[k][...] = gtu_ref[k, :]
    out_refs[K + k][...] = gti_ref[k, :]


_detile = pl.pallas_call(
    _detile_body,
    grid=(_NCHUNK,),
    in_specs=[pl.BlockSpec((K, _CHUNK), lambda c: (0, c))] * 2,
    out_specs=[pl.BlockSpec((_CHUNK,), lambda c: (c,))] * (2 * K),
    out_shape=[jax.ShapeDtypeStruct((_N,), jnp.float32)] * (2 * K),
)


@jax.jit
def kernel(sampleU, sampleI, sampleF, sampleR, alpha,
           betaU, betaI, thetaU, gammaU, gammaI):
  alpha_arr = jnp.full((L,), alpha, dtype=jnp.float32)
  planes = _detile(gammaU.T, gammaI.T)
  gU = planes[:K]
  gI = planes[K:]
  out = _splfm_call(sampleU, sampleI, sampleF, sampleR, alpha_arr,
                    betaU, betaI, thetaU, *gU, *gI)
  return out[0] + out[L]


# speedup vs baseline: 1.2813x; 1.0142x over previous
import functools

import jax
import jax.numpy as jnp
from jax import lax
from jax.experimental import pallas as pl
from jax.experimental.pallas import tpu as pltpu
from jax.experimental.pallas import tpu_sc as plsc

NC = 2
NS = 16
L = 16
B = 16384
K = 16
BPW = B // (NC * NS)
NBLK = BPW // L
SCALE = 0.5 / B

_SC_PARAMS = pltpu.CompilerParams(
    needs_layout_passes=False, use_tc_tiling_on_sc=False)
_MESH = plsc.VectorSubcoreMesh(core_axis_name="c", subcore_axis_name="s")


def _affine_body(sampleU, sampleI, sampleF, sampleR, alpha_arr,
                 betaU, betaI, thetaU, e0_hbm,
                 idxU_v, idxI_v, f_v, r_v, alpha_v,
                 bu_v, bi_v, tu_v, e0_v, sem):
  cid = lax.axis_index("c")
  sid = lax.axis_index("s")
  base = (sid * NC + cid) * BPW

  pltpu.sync_copy(sampleU.at[pl.ds(base, BPW)], idxU_v)
  pltpu.sync_copy(sampleI.at[pl.ds(base, BPW)], idxI_v)
  pltpu.sync_copy(sampleF.at[pl.ds(base, BPW)], f_v)
  pltpu.sync_copy(sampleR.at[pl.ds(base, BPW)], r_v)
  pltpu.sync_copy(alpha_arr, alpha_v)

  cps = [
      pltpu.async_copy(betaU.at[idxU_v], bu_v, sem),
      pltpu.async_copy(betaI.at[idxI_v], bi_v, sem),
      pltpu.async_copy(thetaU.at[idxU_v], tu_v, sem),
  ]
  for cp in cps:
    cp.wait()

  av = alpha_v[...]

  def block_body(blk, carry):
    r0 = blk * L
    e0 = (av + bu_v[pl.ds(r0, L)] + bi_v[pl.ds(r0, L)]
          + tu_v[pl.ds(r0, L)] * f_v[pl.ds(r0, L)] - r_v[pl.ds(r0, L)])
    e0_v[pl.ds(r0, L)] = e0
    return carry

  lax.fori_loop(0, NBLK, block_body, 0)
  pltpu.sync_copy(e0_v, e0_hbm.at[pl.ds(base, BPW)])


_affine_call = functools.partial(
    pl.kernel,
    out_type=jax.ShapeDtypeStruct((B,), jnp.float32),
    mesh=_MESH,
    compiler_params=_SC_PARAMS,
    scratch_types=[
        pltpu.VMEM((BPW,), jnp.int32),
        pltpu.VMEM((BPW,), jnp.int32),
        pltpu.VMEM((BPW,), jnp.float32),
        pltpu.VMEM((BPW,), jnp.float32),
        pltpu.VMEM((L,), jnp.float32),
        pltpu.VMEM((BPW,), jnp.float32),
        pltpu.VMEM((BPW,), jnp.float32),
        pltpu.VMEM((BPW,), jnp.float32),
        pltpu.VMEM((BPW,), jnp.float32),
        pltpu.SemaphoreType.DMA,
    ],
)(_affine_body)


def _dot_body(*refs):
  sampleU, sampleI, e0 = refs[:3]
  gU = refs[3:3 + K]
  gI = refs[3 + K:3 + 2 * K]
  (out_hbm, idxU_v, idxI_v, e0_v, gup_v, gip_v,
   lacc_v, out_v, allp_v, shared, sem) = refs[3 + 2 * K:]

  cid = lax.axis_index("c")
  sid = lax.axis_index("s")
  base = (sid * NC + cid) * BPW

  pltpu.sync_copy(sampleU.at[pl.ds(base, BPW)], idxU_v)
  pltpu.sync_copy(sampleI.at[pl.ds(base, BPW)], idxI_v)
  pltpu.sync_copy(e0.at[pl.ds(base, BPW)], e0_v)

  cps = []
  for k in range(K):
    dst = pl.ds(k * BPW, BPW)
    cps.append(pltpu.async_copy(gU[k].at[idxU_v], gup_v.at[dst], sem))
    cps.append(pltpu.async_copy(gI[k].at[idxI_v], gip_v.at[dst], sem))
  for cp in cps:
    cp.wait()

  def block_body(blk, acc):
    r0 = blk * L
    dot = jnp.zeros((L,), jnp.float32)
    for k in range(K):
      dot = dot + (gup_v[pl.ds(k * BPW + r0, L)] *
                   gip_v[pl.ds(k * BPW + r0, L)])
    e = e0_v[pl.ds(r0, L)] + dot
    return acc + e * e

  loss_acc = lax.fori_loop(0, NBLK, block_body, jnp.zeros((L,), jnp.float32))

  lacc_v[...] = loss_acc
  pltpu.sync_copy(lacc_v, shared.at[sid])
  plsc.subcore_barrier()

  @pl.when(sid == 0)
  def _():
    pltpu.sync_copy(shared, allp_v)
    tot = jnp.zeros((L,), jnp.float32)
    for i in range(NS):
      tot = tot + allp_v[i, :]
    total = jnp.sum(tot) * SCALE
    out_v[...] = jnp.full((L,), total, jnp.float32)
    pltpu.sync_copy(out_v, out_hbm.at[pl.ds(cid * L, L)])


_dot_call = functools.partial(
    pl.kernel,
    out_type=jax.ShapeDtypeStruct((NC * L,), jnp.float32),
    mesh=_MESH,
    compiler_params=_SC_PARAMS,
    scratch_types=[
        pltpu.VMEM((BPW,), jnp.int32),
        pltpu.VMEM((BPW,), jnp.int32),
        pltpu.VMEM((BPW,), jnp.float32),
        pltpu.VMEM((K * BPW,), jnp.float32),
        pltpu.VMEM((K * BPW,), jnp.float32),
        pltpu.VMEM((L,), jnp.float32),
        pltpu.VMEM((L,), jnp.float32),
        pltpu.VMEM((NS, L), jnp.float32),
        pltpu.VMEM_SHARED((NS, L), jnp.float32),
        pltpu.SemaphoreType.DMA,
    ],
)(_dot_body)


_N = 1000000
_CHUNK = 65536
_NCHUNK = -(-_N // _CHUNK)


def _detile_body(gtu_ref, gti_ref, *out_refs):
  for k in range(K):
    out_refs[k][...] = gtu_ref[k, :]
    out_refs[K + k][...] = gti_ref[k, :]


_detile = pl.pallas_call(
    _detile_body,
    grid=(_NCHUNK,),
    in_specs=[pl.BlockSpec((K, _CHUNK), lambda c: (0, c))] * 2,
    out_specs=[pl.BlockSpec((_CHUNK,), lambda c: (c,))] * (2 * K),
    out_shape=[jax.ShapeDtypeStruct((_N,), jnp.float32)] * (2 * K),
)


@jax.jit
def kernel(sampleU, sampleI, sampleF, sampleR, alpha,
           betaU, betaI, thetaU, gammaU, gammaI):
  alpha_arr = jnp.full((L,), alpha, dtype=jnp.float32)
  e0 = _affine_call(sampleU, sampleI, sampleF, sampleR, alpha_arr,
                    betaU, betaI, thetaU)
  planes = _detile(gammaU.T, gammaI.T)
  out = _dot_call(sampleU, sampleI, e0, *planes)
  return out[0] + out[L]
